# Initial kernel scaffold; baseline (speedup 1.0000x reference)
import jax
import jax.numpy as jnp
from jax.experimental import pallas as pl


def kernel(idx, emb_table):
    raise NotImplementedError("write your pallas kernel here")


# speedup vs baseline: 4.6040x; 4.6040x over previous
import functools
import math

import jax
import jax.numpy as jnp
import numpy as np
from jax import lax
from jax.experimental import pallas as pl
from jax.experimental.pallas import tpu as pltpu
from jax.experimental.pallas import tpu_sc as plsc

VOCAB_SIZE = 100000
N_EMBED = 128
BLOCK_SIZE = 200


def _make_pe(block_size, n_embed):
    pe = np.zeros((block_size, n_embed), dtype=np.float32)
    position = np.arange(0, block_size, dtype=np.float32)[:, None]
    div_term = np.exp(
        np.arange(0, n_embed, 2, dtype=np.float32) * (-math.log(10000.0) / n_embed)
    )
    pe[:, 0::2] = np.sin(position * div_term)
    pe[:, 1::2] = np.cos(position * div_term)
    return jnp.asarray(pe)


_NW = 32
_L = 16
_D8 = N_EMBED // _L


def _sc_body(table_hbm, idx_hbm, pe_hbm, out_hbm,
             idx_v, pe_v, rows_v, sem, gsem):
    nc = 2
    wid = lax.axis_index("s") * nc + lax.axis_index("c")
    rows_per_w = idx_v.shape[0]
    n_chunks = rows_per_w // BLOCK_SIZE
    base = wid * rows_per_w

    pltpu.sync_copy(idx_hbm.at[pl.ds(base, rows_per_w)], idx_v)
    pltpu.sync_copy(pe_hbm, pe_v)

    def chunk(c, _):
        row0 = c * BLOCK_SIZE
        s0, s1 = 128, BLOCK_SIZE - 128
        cp0 = pltpu.make_async_copy(
            table_hbm.at[idx_v.at[pl.ds(row0, s0)]],
            rows_v.at[pl.ds(0, s0)], gsem)
        cp1 = pltpu.make_async_copy(
            table_hbm.at[idx_v.at[pl.ds(row0 + s0, s1)]],
            rows_v.at[pl.ds(s0, s1)], gsem)
        cp0.start()
        cp1.start()
        cp0.wait()
        cp1.wait()

        def add_pe(t, _):
            for dv in range(_D8):
                sl = pl.ds(dv * _L, _L)
                rows_v[t, sl] = rows_v[t, sl] + pe_v[t, sl]
            return 0

        lax.fori_loop(0, BLOCK_SIZE, add_pe, 0)
        pltpu.sync_copy(rows_v, out_hbm.at[pl.ds(base + row0, BLOCK_SIZE)])
        return 0

    lax.fori_loop(0, n_chunks, chunk, 0)


@jax.jit
def _embed_sc(idx_flat, emb_table, pe):
    n_rows = idx_flat.shape[0]
    rows_per_w = n_rows // _NW
    fn = pl.kernel(
        _sc_body,
        out_type=jax.ShapeDtypeStruct((n_rows, N_EMBED), jnp.float32),
        mesh=plsc.VectorSubcoreMesh(core_axis_name="c", subcore_axis_name="s"),
        scratch_types=[
            pltpu.VMEM((rows_per_w,), jnp.int32),
            pltpu.VMEM((BLOCK_SIZE, N_EMBED), jnp.float32),
            pltpu.VMEM((BLOCK_SIZE, N_EMBED), jnp.float32),
            pltpu.SemaphoreType.DMA,
            pltpu.SemaphoreType.DMA,
        ],
    )
    return fn(emb_table, idx_flat, pe)


def kernel(idx, emb_table):
    B, T = idx.shape
    pe = _make_pe(BLOCK_SIZE, N_EMBED)[:T]
    idx_flat = idx.reshape(-1).astype(jnp.int32)
    out = _embed_sc(idx_flat, emb_table, pe)
    return out.reshape(B, T, N_EMBED)
